# Initial kernel scaffold; baseline (speedup 1.0000x reference)
import jax
import jax.numpy as jnp
from jax.experimental import pallas as pl


def kernel(x_flat, params):
    raise NotImplementedError("write your pallas kernel here")


# speedup vs baseline: 4.3858x; 4.3858x over previous
import functools

import jax
import jax.numpy as jnp
from jax import lax
from jax.experimental import pallas as pl
from jax.experimental.pallas import tpu as pltpu

N_PART = 13
NP = 16
S_DIM = 3
HID = 64
N_LAYERS = 5
BS = 32


def _lnk(x, g, b):
    m = jnp.mean(x, axis=-1, keepdims=True)
    v = jnp.mean((x - m) ** 2, axis=-1, keepdims=True)
    return (x - m) / jnp.sqrt(v + 1e-5) * g + b


def _silu(x):
    return x * jax.nn.sigmoid(x)


def _egnn_kernel(coord_ref, h0_ref, eW1a_ref, eW1b_ref, wr_ref, we_ref,
                 eb1_ref, eg1_ref, ebe1_ref, eW2_ref, eb2_ref, eg2_ref,
                 ebe2_ref, aw_ref, ab_ref, cW1_ref, cb1_ref, cg1_ref,
                 cbe1_ref, cw2_ref, nW1a_ref, nW1b_ref, nb1_ref, ng1_ref,
                 nbe1_ref, nW2_ref, nb2_ref, out_ref):
    bs = coord_ref.shape[0]
    R = bs * NP * NP
    f32 = jnp.float32

    coord = coord_ref[...]
    x_init = coord
    h = jnp.broadcast_to(h0_ref[...], (bs * NP, HID))

    ii = lax.broadcasted_iota(jnp.int32, (1, NP, NP, 1), 1)
    jj = lax.broadcasted_iota(jnp.int32, (1, NP, NP, 1), 2)
    emask = ((ii != jj) & (ii < N_PART) & (jj < N_PART)).astype(f32)

    cd0 = coord[:, :, None, :] - coord[:, None, :, :]
    edge_attr = jnp.sum(cd0 * cd0, axis=-1, keepdims=True).reshape(R, 1)

    def dot(x, w):
        return jnp.dot(x, w, preferred_element_type=f32)

    for l in range(N_LAYERS):
        cd = (coord[:, :, None, :] - coord[:, None, :, :]).reshape(R, S_DIM)
        radial = jnp.sum(cd * cd, axis=-1, keepdims=True)
        cdn = cd / (jnp.sqrt(radial + 1e-8) + 1.0)

        a = dot(h, eW1a_ref[l])
        c = dot(h, eW1b_ref[l])
        pre = (a.reshape(bs, NP, 1, HID)
               + c.reshape(bs, 1, NP, HID)).reshape(R, HID)
        pre = pre + radial * wr_ref[l] + edge_attr * we_ref[l] + eb1_ref[l]
        m = _silu(_lnk(pre, eg1_ref[l], ebe1_ref[l]))
        m = _silu(_lnk(dot(m, eW2_ref[l]) + eb2_ref[l], eg2_ref[l],
                       ebe2_ref[l]))
        att = jax.nn.sigmoid(
            jnp.sum(m * aw_ref[l], axis=-1, keepdims=True) + ab_ref[l])
        ef = m * att

        c1 = _silu(_lnk(dot(ef, cW1_ref[l]) + cb1_ref[l], cg1_ref[l],
                        cbe1_ref[l]))
        c2 = jnp.sum(c1 * cw2_ref[l], axis=-1, keepdims=True)

        trans = (cdn * c2).reshape(bs, NP, NP, S_DIM) * emask
        coord = coord + jnp.sum(trans, axis=2)

        agg = jnp.sum(ef.reshape(bs, NP, NP, HID) * emask,
                      axis=2).reshape(bs * NP, HID)
        o = _silu(_lnk(dot(h, nW1a_ref[l]) + dot(agg, nW1b_ref[l])
                       + nb1_ref[l], ng1_ref[l], nbe1_ref[l]))
        h = h + dot(o, nW2_ref[l]) + nb2_ref[l]

    nmask = (lax.broadcasted_iota(jnp.int32, (1, NP, 1), 1)
             < N_PART).astype(f32)
    vel = (coord - x_init) * nmask
    mean = jnp.sum(vel, axis=1, keepdims=True) * (1.0 / N_PART)
    out_ref[...] = (vel - mean) * nmask


@jax.jit
def kernel(x_flat, params):
    B = x_flat.shape[0]
    coord0 = x_flat.reshape(B, N_PART, S_DIM)
    coord0 = jnp.pad(coord0, ((0, 0), (0, NP - N_PART), (0, 0)))
    h0 = (params['emb_W'][:, 0] + params['emb_b'])[None, :]

    L = params['layers']

    def st(f):
        return jnp.stack([f(p) for p in L])

    ops = [
        coord0,
        h0,
        st(lambda p: p['e_W1'][:, :HID].T),
        st(lambda p: p['e_W1'][:, HID:2 * HID].T),
        st(lambda p: p['e_W1'][:, 2 * HID][None, :]),
        st(lambda p: p['e_W1'][:, 2 * HID + 1][None, :]),
        st(lambda p: p['e_b1'][None, :]),
        st(lambda p: p['e_g1'][None, :]),
        st(lambda p: p['e_be1'][None, :]),
        st(lambda p: p['e_W2'].T),
        st(lambda p: p['e_b2'][None, :]),
        st(lambda p: p['e_g2'][None, :]),
        st(lambda p: p['e_be2'][None, :]),
        st(lambda p: p['a_W'][0][None, :]),
        st(lambda p: p['a_b'][None, :]),
        st(lambda p: p['c_W1'].T),
        st(lambda p: p['c_b1'][None, :]),
        st(lambda p: p['c_g1'][None, :]),
        st(lambda p: p['c_be1'][None, :]),
        st(lambda p: p['c_W2'][0][None, :]),
        st(lambda p: p['n_W1'][:, :HID].T),
        st(lambda p: p['n_W1'][:, HID:].T),
        st(lambda p: p['n_b1'][None, :]),
        st(lambda p: p['n_g1'][None, :]),
        st(lambda p: p['n_be1'][None, :]),
        st(lambda p: p['n_W2'].T),
        st(lambda p: p['n_b2'][None, :]),
    ]

    in_specs = [pl.BlockSpec((BS, NP, S_DIM), lambda i: (i, 0, 0))]
    for op in ops[1:]:
        shp = op.shape
        in_specs.append(
            pl.BlockSpec(shp, functools.partial(
                lambda nd, i: (0,) * nd, len(shp))))

    out = pl.pallas_call(
        _egnn_kernel,
        grid=(B // BS,),
        in_specs=in_specs,
        out_specs=pl.BlockSpec((BS, NP, S_DIM), lambda i: (i, 0, 0)),
        out_shape=jax.ShapeDtypeStruct((B, NP, S_DIM), jnp.float32),
        compiler_params=pltpu.CompilerParams(
            dimension_semantics=("parallel",)),
    )(*ops)

    return out[:, :N_PART, :].reshape(B, N_PART * S_DIM)


# speedup vs baseline: 4.5645x; 1.0408x over previous
import functools

import jax
import jax.numpy as jnp
from jax import lax
from jax.experimental import pallas as pl
from jax.experimental.pallas import tpu as pltpu

N_PART = 13
NP = 16
S_DIM = 3
HID = 64
N_LAYERS = 5
BS = 64


def _lnk(x, g, b):
    m = jnp.mean(x, axis=-1, keepdims=True)
    v = jnp.mean((x - m) ** 2, axis=-1, keepdims=True)
    return (x - m) / jnp.sqrt(v + 1e-5) * g + b


def _silu(x):
    return x * jax.nn.sigmoid(x)


def _egnn_kernel(coord_ref, h0_ref, eW1ab_ref, wr_ref, we_ref,
                 eb1_ref, eg1_ref, ebe1_ref, eW2_ref, eb2_ref, eg2_ref,
                 ebe2_ref, aw_ref, ab_ref, cW1_ref, cb1_ref, cg1_ref,
                 cbe1_ref, cw2_ref, nW1_ref, nb1_ref, ng1_ref,
                 nbe1_ref, nW2_ref, nb2_ref, out_ref):
    bs = coord_ref.shape[0]
    R = bs * NP * NP
    f32 = jnp.float32

    coord = coord_ref[...]
    x_init = coord
    h = jnp.broadcast_to(h0_ref[...], (bs * NP, HID))

    ii = lax.broadcasted_iota(jnp.int32, (1, NP, NP, 1), 1)
    jj = lax.broadcasted_iota(jnp.int32, (1, NP, NP, 1), 2)
    emask = ((ii != jj) & (ii < N_PART) & (jj < N_PART)).astype(f32)

    cd0 = coord[:, :, None, :] - coord[:, None, :, :]
    edge_attr = jnp.sum(cd0 * cd0, axis=-1, keepdims=True).reshape(R, 1)

    def dot(x, w):
        return jnp.dot(x, w, preferred_element_type=f32)

    for l in range(N_LAYERS):
        cd = (coord[:, :, None, :] - coord[:, None, :, :]).reshape(R, S_DIM)
        radial = jnp.sum(cd * cd, axis=-1, keepdims=True)
        cdn = cd / (jnp.sqrt(radial + 1e-8) + 1.0)

        ac = dot(h, eW1ab_ref[l])
        a = ac[:, :HID]
        c = ac[:, HID:]
        pre = (a.reshape(bs, NP, 1, HID)
               + c.reshape(bs, 1, NP, HID)).reshape(R, HID)
        pre = pre + radial * wr_ref[l] + edge_attr * we_ref[l] + eb1_ref[l]
        m = _silu(_lnk(pre, eg1_ref[l], ebe1_ref[l]))
        m = _silu(_lnk(dot(m, eW2_ref[l]) + eb2_ref[l], eg2_ref[l],
                       ebe2_ref[l]))
        att = jax.nn.sigmoid(
            jnp.sum(m * aw_ref[l], axis=-1, keepdims=True) + ab_ref[l])
        ef = m * att

        c1 = _silu(_lnk(dot(ef, cW1_ref[l]) + cb1_ref[l], cg1_ref[l],
                        cbe1_ref[l]))
        c2 = jnp.sum(c1 * cw2_ref[l], axis=-1, keepdims=True)

        trans = (cdn * c2).reshape(bs, NP, NP, S_DIM) * emask
        coord = coord + jnp.sum(trans, axis=2)

        agg = jnp.sum(ef.reshape(bs, NP, NP, HID) * emask,
                      axis=2).reshape(bs * NP, HID)
        hagg = jnp.concatenate([h, agg], axis=-1)
        o = _silu(_lnk(dot(hagg, nW1_ref[l]) + nb1_ref[l],
                       ng1_ref[l], nbe1_ref[l]))
        h = h + dot(o, nW2_ref[l]) + nb2_ref[l]

    nmask = (lax.broadcasted_iota(jnp.int32, (1, NP, 1), 1)
             < N_PART).astype(f32)
    vel = (coord - x_init) * nmask
    mean = jnp.sum(vel, axis=1, keepdims=True) * (1.0 / N_PART)
    out_ref[...] = (vel - mean) * nmask


@jax.jit
def kernel(x_flat, params):
    B = x_flat.shape[0]
    coord0 = x_flat.reshape(B, N_PART, S_DIM)
    coord0 = jnp.pad(coord0, ((0, 0), (0, NP - N_PART), (0, 0)))
    h0 = (params['emb_W'][:, 0] + params['emb_b'])[None, :]

    L = params['layers']

    def st(f):
        return jnp.stack([f(p) for p in L])

    ops = [
        coord0,
        h0,
        st(lambda p: jnp.concatenate(
            [p['e_W1'][:, :HID].T, p['e_W1'][:, HID:2 * HID].T], axis=-1)),
        st(lambda p: p['e_W1'][:, 2 * HID][None, :]),
        st(lambda p: p['e_W1'][:, 2 * HID + 1][None, :]),
        st(lambda p: p['e_b1'][None, :]),
        st(lambda p: p['e_g1'][None, :]),
        st(lambda p: p['e_be1'][None, :]),
        st(lambda p: p['e_W2'].T),
        st(lambda p: p['e_b2'][None, :]),
        st(lambda p: p['e_g2'][None, :]),
        st(lambda p: p['e_be2'][None, :]),
        st(lambda p: p['a_W'][0][None, :]),
        st(lambda p: p['a_b'][None, :]),
        st(lambda p: p['c_W1'].T),
        st(lambda p: p['c_b1'][None, :]),
        st(lambda p: p['c_g1'][None, :]),
        st(lambda p: p['c_be1'][None, :]),
        st(lambda p: p['c_W2'][0][None, :]),
        st(lambda p: p['n_W1'].T),
        st(lambda p: p['n_b1'][None, :]),
        st(lambda p: p['n_g1'][None, :]),
        st(lambda p: p['n_be1'][None, :]),
        st(lambda p: p['n_W2'].T),
        st(lambda p: p['n_b2'][None, :]),
    ]

    in_specs = [pl.BlockSpec((BS, NP, S_DIM), lambda i: (i, 0, 0))]
    for op in ops[1:]:
        shp = op.shape
        in_specs.append(
            pl.BlockSpec(shp, functools.partial(
                lambda nd, i: (0,) * nd, len(shp))))

    out = pl.pallas_call(
        _egnn_kernel,
        grid=(B // BS,),
        in_specs=in_specs,
        out_specs=pl.BlockSpec((BS, NP, S_DIM), lambda i: (i, 0, 0)),
        out_shape=jax.ShapeDtypeStruct((B, NP, S_DIM), jnp.float32),
        compiler_params=pltpu.CompilerParams(
            dimension_semantics=("parallel",)),
    )(*ops)

    return out[:, :N_PART, :].reshape(B, N_PART * S_DIM)


# speedup vs baseline: 5.6649x; 1.2411x over previous
import functools

import jax
import jax.numpy as jnp
from jax import lax
from jax.experimental import pallas as pl
from jax.experimental.pallas import tpu as pltpu

N_PART = 13
NP = 16
S_DIM = 3
HID = 64
N_LAYERS = 5
BS = 64


def _lnk(x, g, b):
    m = jnp.mean(x, axis=-1, keepdims=True)
    v = jnp.mean((x - m) ** 2, axis=-1, keepdims=True)
    return (x - m) / jnp.sqrt(v + 1e-5) * g + b


def _silu(x):
    return x * jax.nn.sigmoid(x)


def _egnn_kernel(coord_ref, h0_ref, eW1ab_ref, wr_ref, we_ref,
                 eb1_ref, eg1_ref, ebe1_ref, eW2_ref, eb2_ref, eg2_ref,
                 ebe2_ref, aw_ref, ab_ref, cW1_ref, cb1_ref, cg1_ref,
                 cbe1_ref, cw2_ref, nW1_ref, nb1_ref, ng1_ref,
                 nbe1_ref, nW2_ref, nb2_ref, out_ref):
    bs = coord_ref.shape[0]
    NI = N_PART
    R = bs * NI * NP
    f32 = jnp.float32

    coord = coord_ref[...]
    x_init = coord
    h = jnp.broadcast_to(h0_ref[...], (bs * NP, HID))

    ii = lax.broadcasted_iota(jnp.int32, (1, NI, NP, 1), 1)
    jj = lax.broadcasted_iota(jnp.int32, (1, NI, NP, 1), 2)
    mcol = jnp.broadcast_to(
        ((ii != jj) & (jj < N_PART)).astype(f32),
        (bs, NI, NP, 1)).reshape(R, 1)

    def pair_diff(co):
        return (co[:, :NI, None, :] - co[:, None, :, :]).reshape(R, S_DIM)

    cd0 = pair_diff(coord)
    edge_attr = jnp.sum(cd0 * cd0, axis=-1, keepdims=True)

    def dot(x, w):
        return jnp.dot(x, w, preferred_element_type=f32)

    zpad3 = jnp.zeros((bs, NP - NI, S_DIM), f32)
    zpadh = jnp.zeros((bs, NP - NI, HID), f32)

    for l in range(N_LAYERS):
        cd = pair_diff(coord)
        radial = jnp.sum(cd * cd, axis=-1, keepdims=True)
        cdn = cd / (jnp.sqrt(radial + 1e-8) + 1.0)

        ac = dot(h, eW1ab_ref[l])
        a = (ac[:, :HID] + eb1_ref[l]).reshape(bs, NP, 1, HID)[:, :NI]
        c = ac[:, HID:].reshape(bs, 1, NP, HID)
        pre = (a + c).reshape(R, HID)
        pre = pre + radial * wr_ref[l] + edge_attr * we_ref[l]
        m = _silu(_lnk(pre, eg1_ref[l], ebe1_ref[l]))
        m = _silu(_lnk(dot(m, eW2_ref[l]) + eb2_ref[l], eg2_ref[l],
                       ebe2_ref[l]))
        att = jax.nn.sigmoid(
            jnp.sum(m * aw_ref[l], axis=-1, keepdims=True) + ab_ref[l])
        ef = m * (att * mcol)

        c1 = _silu(_lnk(dot(ef, cW1_ref[l]) + cb1_ref[l], cg1_ref[l],
                        cbe1_ref[l]))
        c2 = jnp.sum(c1 * cw2_ref[l], axis=-1, keepdims=True) * mcol

        dlt = jnp.sum((cdn * c2).reshape(bs, NI, NP, S_DIM), axis=2)
        coord = coord + jnp.concatenate([dlt, zpad3], axis=1)

        agg = jnp.sum(ef.reshape(bs, NI, NP, HID), axis=2)
        agg = jnp.concatenate([agg, zpadh], axis=1).reshape(bs * NP, HID)
        hagg = jnp.concatenate([h, agg], axis=-1)
        o = _silu(_lnk(dot(hagg, nW1_ref[l]) + nb1_ref[l],
                       ng1_ref[l], nbe1_ref[l]))
        h = h + dot(o, nW2_ref[l]) + nb2_ref[l]

    nmask = (lax.broadcasted_iota(jnp.int32, (1, NP, 1), 1)
             < N_PART).astype(f32)
    vel = (coord - x_init) * nmask
    mean = jnp.sum(vel, axis=1, keepdims=True) * (1.0 / N_PART)
    out_ref[...] = (vel - mean) * nmask


@jax.jit
def kernel(x_flat, params):
    B = x_flat.shape[0]
    coord0 = x_flat.reshape(B, N_PART, S_DIM)
    coord0 = jnp.pad(coord0, ((0, 0), (0, NP - N_PART), (0, 0)))
    h0 = (params['emb_W'][:, 0] + params['emb_b'])[None, :]

    L = params['layers']

    def st(f):
        return jnp.stack([f(p) for p in L])

    ops = [
        coord0,
        h0,
        st(lambda p: jnp.concatenate(
            [p['e_W1'][:, :HID].T, p['e_W1'][:, HID:2 * HID].T], axis=-1)),
        st(lambda p: p['e_W1'][:, 2 * HID][None, :]),
        st(lambda p: p['e_W1'][:, 2 * HID + 1][None, :]),
        st(lambda p: p['e_b1'][None, :]),
        st(lambda p: p['e_g1'][None, :]),
        st(lambda p: p['e_be1'][None, :]),
        st(lambda p: p['e_W2'].T),
        st(lambda p: p['e_b2'][None, :]),
        st(lambda p: p['e_g2'][None, :]),
        st(lambda p: p['e_be2'][None, :]),
        st(lambda p: p['a_W'][0][None, :]),
        st(lambda p: p['a_b'][None, :]),
        st(lambda p: p['c_W1'].T),
        st(lambda p: p['c_b1'][None, :]),
        st(lambda p: p['c_g1'][None, :]),
        st(lambda p: p['c_be1'][None, :]),
        st(lambda p: p['c_W2'][0][None, :]),
        st(lambda p: p['n_W1'].T),
        st(lambda p: p['n_b1'][None, :]),
        st(lambda p: p['n_g1'][None, :]),
        st(lambda p: p['n_be1'][None, :]),
        st(lambda p: p['n_W2'].T),
        st(lambda p: p['n_b2'][None, :]),
    ]

    in_specs = [pl.BlockSpec((BS, NP, S_DIM), lambda i: (i, 0, 0))]
    for op in ops[1:]:
        shp = op.shape
        in_specs.append(
            pl.BlockSpec(shp, functools.partial(
                lambda nd, i: (0,) * nd, len(shp))))

    out = pl.pallas_call(
        _egnn_kernel,
        grid=(B // BS,),
        in_specs=in_specs,
        out_specs=pl.BlockSpec((BS, NP, S_DIM), lambda i: (i, 0, 0)),
        out_shape=jax.ShapeDtypeStruct((B, NP, S_DIM), jnp.float32),
        compiler_params=pltpu.CompilerParams(
            dimension_semantics=("parallel",)),
    )(*ops)

    return out[:, :N_PART, :].reshape(B, N_PART * S_DIM)
